# baseline (speedup 1.0000x reference)
import jax
import jax.numpy as jnp
from jax import lax
from jax.experimental import pallas as pl
from jax.experimental.pallas import tpu as pltpu
from jax.experimental.pallas import tpu_sc as plsc

BATCH = 16384
EMBED_DIM = 16
NUM_ROWS = 1000000
SLAB = 8
NUM_SLABS = NUM_ROWS // SLAB
DATA_SLABS = BATCH // SLAB
NUM_CORES = 2
NUM_SUBCORES = 16
NUM_WORKERS = NUM_CORES * NUM_SUBCORES
BPW = BATCH // NUM_WORKERS
SPW = BPW // SLAB
LANES = 16
NCH = BPW // LANES
STAGE = 16


def _body(data_hbm, user_hbm, item_hbm, out_hbm,
          dstage, u_all, i_all, ub0, ub1, ib0, ib1, out_v, sem0, sem1):
    u_bufs = (ub0, ub1)
    i_bufs = (ib0, ib1)
    sems = (sem0, sem1)

    wid = lax.axis_index("s") * NUM_CORES + lax.axis_index("c")
    base = wid * BPW
    slab0 = wid * SPW

    lanes = lax.iota(jnp.int32, 16)

    for s in range(SPW // STAGE):
        pltpu.sync_copy(data_hbm.at[pl.ds(slab0 + s * STAGE, STAGE)], dstage)
        for g in range(STAGE * SLAB // LANES):
            row = g * LANES + lanes
            sl = lax.shift_right_logical(row, 3)
            sr = lax.bitwise_and(row, 7)
            u16 = plsc.load_gather(dstage, [sl, sr, jnp.zeros((16,), jnp.int32)])
            i16 = plsc.load_gather(dstage, [sl, sr, jnp.ones((16,), jnp.int32)])
            u_all[pl.ds(s * STAGE * SLAB + g * LANES, LANES)] = u16
            i_all[pl.ds(s * STAGE * SLAB + g * LANES, LANES)] = i16

    def fire(c, par):
        iu = plsc.load_gather(u_all, [c * LANES + lanes])
        ii = plsc.load_gather(i_all, [c * LANES + lanes])
        for j in range(LANES):
            ru = iu[j]
            ri = ii[j]
            pltpu.async_copy(
                user_hbm.at[ru >> 3, ru & 7], u_bufs[par].at[j], sems[par])
            pltpu.async_copy(
                item_hbm.at[ri >> 3, ri & 7], i_bufs[par].at[j], sems[par])

    def drain(par):
        for j in range(LANES):
            pltpu.make_async_copy(
                user_hbm.at[0, 0], u_bufs[par].at[j], sems[par]).wait()
            pltpu.make_async_copy(
                item_hbm.at[0, 0], i_bufs[par].at[j], sems[par]).wait()

    def compute(c, par):
        acc = jnp.zeros((16,), jnp.float32)
        for d in range(EMBED_DIM):
            col = jnp.full((16,), d, jnp.int32)
            acc = acc + (plsc.load_gather(u_bufs[par], [lanes, col]) *
                         plsc.load_gather(i_bufs[par], [lanes, col]))
        out_v[pl.ds(c * LANES, LANES)] = 1.0 / (1.0 + jnp.exp(-acc))

    fire(0, 0)

    def pair(j, _):
        a = 2 * j
        fire(a + 1, 1)
        drain(0)
        compute(a, 0)

        @pl.when(j < NCH // 2 - 1)
        def _():
            fire(a + 2, 0)

        drain(1)
        compute(a + 1, 1)
        return 0

    lax.fori_loop(0, NCH // 2, pair, 0)

    pltpu.sync_copy(out_v, out_hbm.at[pl.ds(base, BPW)])


@jax.jit
def _run(data_slabs, user_slabs, item_slabs):
    mesh = plsc.VectorSubcoreMesh(
        core_axis_name="c", subcore_axis_name="s",
        num_cores=NUM_CORES, num_subcores=NUM_SUBCORES)
    scratch = [
        pltpu.VMEM((STAGE, SLAB, 2), jnp.int32),
        pltpu.VMEM((BPW,), jnp.int32),
        pltpu.VMEM((BPW,), jnp.int32),
        pltpu.VMEM((LANES, EMBED_DIM), jnp.float32),
        pltpu.VMEM((LANES, EMBED_DIM), jnp.float32),
        pltpu.VMEM((LANES, EMBED_DIM), jnp.float32),
        pltpu.VMEM((LANES, EMBED_DIM), jnp.float32),
        pltpu.VMEM((BPW,), jnp.float32),
        pltpu.SemaphoreType.DMA,
        pltpu.SemaphoreType.DMA,
    ]
    f = pl.kernel(
        _body,
        out_type=jax.ShapeDtypeStruct((BATCH,), jnp.float32),
        mesh=mesh,
        scratch_types=scratch,
        compiler_params=pltpu.CompilerParams(
            needs_layout_passes=False, use_tc_tiling_on_sc=False),
    )
    return f(data_slabs, user_slabs, item_slabs)


def kernel(data, user_embeds, item_embeds):
    data_slabs = data.astype(jnp.int32).reshape(DATA_SLABS, SLAB, 2)
    user_slabs = user_embeds.reshape(NUM_SLABS, SLAB, EMBED_DIM)
    item_slabs = item_embeds.reshape(NUM_SLABS, SLAB, EMBED_DIM)
    return _run(data_slabs, user_slabs, item_slabs)


# speedup vs baseline: 2.6583x; 2.6583x over previous
import jax
import jax.numpy as jnp
from jax import lax
from jax.experimental import pallas as pl
from jax.experimental.pallas import tpu as pltpu
from jax.experimental.pallas import tpu_sc as plsc

BATCH = 16384
EMBED_DIM = 16
NUM_ROWS = 1000000
SLAB = 8
NUM_SLABS = NUM_ROWS // SLAB
DATA_SLABS = BATCH // SLAB
NUM_CORES = 2
NUM_SUBCORES = 16
NUM_WORKERS = NUM_CORES * NUM_SUBCORES
BPW = BATCH // NUM_WORKERS
SPW = BPW // SLAB
LANES = 16
NCH = BPW // LANES
STAGE = 16


def _body(data_hbm, user_hbm, item_hbm, out_hbm,
          dstage, u_all, i_all, ub0, ub1, ib0, ib1, out_v, sem0, sem1):
    u_bufs = (ub0, ub1)
    i_bufs = (ib0, ib1)
    sems = (sem0, sem1)

    wid = lax.axis_index("s") * NUM_CORES + lax.axis_index("c")
    base = wid * BPW
    slab0 = wid * SPW

    lanes = lax.iota(jnp.int32, 16)

    for s in range(SPW // STAGE):
        pltpu.sync_copy(data_hbm.at[pl.ds(slab0 + s * STAGE, STAGE)], dstage)
        for g in range(STAGE * SLAB // LANES):
            row = g * LANES + lanes
            sl = lax.shift_right_logical(row, 3)
            sr = lax.bitwise_and(row, 7)
            u16 = plsc.load_gather(dstage, [sl, sr, jnp.zeros((16,), jnp.int32)])
            i16 = plsc.load_gather(dstage, [sl, sr, jnp.ones((16,), jnp.int32)])
            u_all[pl.ds(s * STAGE * SLAB + g * LANES, LANES)] = u16
            i_all[pl.ds(s * STAGE * SLAB + g * LANES, LANES)] = i16

    def fire(c, par):
        iu = plsc.load_gather(u_all, [c * LANES + lanes])
        ii = plsc.load_gather(i_all, [c * LANES + lanes])
        for j in range(LANES):
            ru = iu[j]
            ri = ii[j]
            pltpu.async_copy(
                user_hbm.at[ru >> 3, ru & 7], u_bufs[par].at[j], sems[par])
            pltpu.async_copy(
                item_hbm.at[ri >> 3, ri & 7], i_bufs[par].at[j], sems[par])

    def drain(par):
        for j in range(LANES):
            pltpu.make_async_copy(
                user_hbm.at[0, 0], u_bufs[par].at[j], sems[par]).wait()
            pltpu.make_async_copy(
                item_hbm.at[0, 0], i_bufs[par].at[j], sems[par]).wait()

    def compute(c, par):
        acc = jnp.zeros((16,), jnp.float32)
        for d in range(EMBED_DIM):
            col = jnp.full((16,), d, jnp.int32)
            acc = acc + (plsc.load_gather(u_bufs[par], [lanes, col]) *
                         plsc.load_gather(i_bufs[par], [lanes, col]))
        out_v[pl.ds(c * LANES, LANES)] = 1.0 / (1.0 + jnp.exp(-acc))

    fire(0, 0)

    def pair(j, _):
        a = 2 * j
        fire(a + 1, 1)
        drain(0)
        compute(a, 0)

        @pl.when(j < NCH // 2 - 1)
        def _():
            fire(a + 2, 0)

        drain(1)
        compute(a + 1, 1)
        return 0

    lax.fori_loop(0, NCH // 2, pair, 0)

    pltpu.sync_copy(out_v, out_hbm.at[pl.ds(base, BPW)])


@jax.jit
def _run(data_slabs, user_slabs, item_slabs):
    mesh = plsc.VectorSubcoreMesh(
        core_axis_name="c", subcore_axis_name="s",
        num_cores=NUM_CORES, num_subcores=NUM_SUBCORES)
    scratch = [
        pltpu.VMEM((STAGE, SLAB, 2), jnp.int32),
        pltpu.VMEM((BPW,), jnp.int32),
        pltpu.VMEM((BPW,), jnp.int32),
        pltpu.VMEM((LANES, EMBED_DIM), jnp.float32),
        pltpu.VMEM((LANES, EMBED_DIM), jnp.float32),
        pltpu.VMEM((LANES, EMBED_DIM), jnp.float32),
        pltpu.VMEM((LANES, EMBED_DIM), jnp.float32),
        pltpu.VMEM((BPW,), jnp.float32),
        pltpu.SemaphoreType.DMA,
        pltpu.SemaphoreType.DMA,
    ]
    f = pl.kernel(
        _body,
        out_type=jax.ShapeDtypeStruct((BATCH,), jnp.float32),
        mesh=mesh,
        scratch_types=scratch,
        compiler_params=pltpu.CompilerParams(
            needs_layout_passes=False, use_tc_tiling_on_sc=True),
    )
    return f(data_slabs, user_slabs, item_slabs)


def kernel(data, user_embeds, item_embeds):
    data_slabs = data.astype(jnp.int32).reshape(DATA_SLABS, SLAB, 2)
    user_slabs = user_embeds.reshape(NUM_SLABS, SLAB, EMBED_DIM)
    item_slabs = item_embeds.reshape(NUM_SLABS, SLAB, EMBED_DIM)
    return _run(data_slabs, user_slabs, item_slabs)


# speedup vs baseline: 2.6712x; 1.0048x over previous
import jax
import jax.numpy as jnp
from jax import lax
from jax.experimental import pallas as pl
from jax.experimental.pallas import tpu as pltpu
from jax.experimental.pallas import tpu_sc as plsc

BATCH = 16384
EMBED_DIM = 16
NUM_ROWS = 1000000
SLAB = 8
NUM_SLABS = NUM_ROWS // SLAB
DATA_SLABS = BATCH // SLAB
NUM_CORES = 2
NUM_SUBCORES = 16
NUM_WORKERS = NUM_CORES * NUM_SUBCORES
BPW = BATCH // NUM_WORKERS
SPW = BPW // SLAB
LANES = 16
NCH = BPW // LANES
STAGE = 16


NPAR = 4


def _body(data_hbm, user_hbm, item_hbm, out_hbm,
          dstage, u_all, i_all, ub0, ub1, ub2, ub3, ib0, ib1, ib2, ib3,
          out_v, sem0, sem1, sem2, sem3):
    u_bufs = (ub0, ub1, ub2, ub3)
    i_bufs = (ib0, ib1, ib2, ib3)
    sems = (sem0, sem1, sem2, sem3)

    wid = lax.axis_index("s") * NUM_CORES + lax.axis_index("c")
    base = wid * BPW
    slab0 = wid * SPW

    lanes = lax.iota(jnp.int32, 16)

    for s in range(SPW // STAGE):
        pltpu.sync_copy(data_hbm.at[pl.ds(slab0 + s * STAGE, STAGE)], dstage)
        for g in range(STAGE * SLAB // LANES):
            row = g * LANES + lanes
            sl = lax.shift_right_logical(row, 3)
            sr = lax.bitwise_and(row, 7)
            u16 = plsc.load_gather(dstage, [sl, sr, jnp.zeros((16,), jnp.int32)])
            i16 = plsc.load_gather(dstage, [sl, sr, jnp.ones((16,), jnp.int32)])
            u_all[pl.ds(s * STAGE * SLAB + g * LANES, LANES)] = u16
            i_all[pl.ds(s * STAGE * SLAB + g * LANES, LANES)] = i16

    def fire(c, par):
        iu = plsc.load_gather(u_all, [c * LANES + lanes])
        ii = plsc.load_gather(i_all, [c * LANES + lanes])
        for j in range(LANES):
            ru = iu[j]
            ri = ii[j]
            pltpu.async_copy(
                user_hbm.at[ru >> 3, ru & 7], u_bufs[par].at[j], sems[par])
            pltpu.async_copy(
                item_hbm.at[ri >> 3, ri & 7], i_bufs[par].at[j], sems[par])

    def drain(par):
        for j in range(LANES):
            pltpu.make_async_copy(
                user_hbm.at[0, 0], u_bufs[par].at[j], sems[par]).wait()
            pltpu.make_async_copy(
                item_hbm.at[0, 0], i_bufs[par].at[j], sems[par]).wait()

    def compute(c, par):
        acc = jnp.zeros((16,), jnp.float32)
        for d in range(EMBED_DIM):
            col = jnp.full((16,), d, jnp.int32)
            acc = acc + (plsc.load_gather(u_bufs[par], [lanes, col]) *
                         plsc.load_gather(i_bufs[par], [lanes, col]))
        out_v[pl.ds(c * LANES, LANES)] = 1.0 / (1.0 + jnp.exp(-acc))

    for p in range(NPAR - 1):
        fire(p, p)

    def quad(j, _):
        a = NPAR * j
        fire(a + NPAR - 1, NPAR - 1)
        for p in range(NPAR):
            drain(p)
            compute(a + p, p)

            @pl.when(j < NCH // NPAR - 1)
            def _(p=p):
                fire(a + NPAR + p, p)

        return 0

    lax.fori_loop(0, NCH // NPAR, quad, 0)

    pltpu.sync_copy(out_v, out_hbm.at[pl.ds(base, BPW)])


@jax.jit
def _run(data_slabs, user_slabs, item_slabs):
    mesh = plsc.VectorSubcoreMesh(
        core_axis_name="c", subcore_axis_name="s",
        num_cores=NUM_CORES, num_subcores=NUM_SUBCORES)
    scratch = [
        pltpu.VMEM((STAGE, SLAB, 2), jnp.int32),
        pltpu.VMEM((BPW,), jnp.int32),
        pltpu.VMEM((BPW,), jnp.int32),
    ]
    scratch += [pltpu.VMEM((LANES, EMBED_DIM), jnp.float32)
                for _ in range(2 * NPAR)]
    scratch += [pltpu.VMEM((BPW,), jnp.float32)]
    scratch += [pltpu.SemaphoreType.DMA for _ in range(NPAR)]
    f = pl.kernel(
        _body,
        out_type=jax.ShapeDtypeStruct((BATCH,), jnp.float32),
        mesh=mesh,
        scratch_types=scratch,
        compiler_params=pltpu.CompilerParams(
            needs_layout_passes=False, use_tc_tiling_on_sc=True),
    )
    return f(data_slabs, user_slabs, item_slabs)


def kernel(data, user_embeds, item_embeds):
    data_slabs = data.astype(jnp.int32).reshape(DATA_SLABS, SLAB, 2)
    user_slabs = user_embeds.reshape(NUM_SLABS, SLAB, EMBED_DIM)
    item_slabs = item_embeds.reshape(NUM_SLABS, SLAB, EMBED_DIM)
    return _run(data_slabs, user_slabs, item_slabs)
